# baseline (speedup 1.0000x reference)
import functools

import jax
import jax.numpy as jnp
from jax import lax
from jax.experimental import pallas as pl
from jax.experimental.pallas import tpu as pltpu
from jax.experimental.pallas import tpu_sc as plsc

_NC = 2
_NS = 16
_NW = _NC * _NS


def _tc_combo(voc: int, d: int, tx: int, vp: int):
    def body(tok_ref, pos_ref, w_ref, b_ref, out_ref, tokl_scr):
        t = pl.program_id(0)

        @pl.when(t == 0)
        def _():
            tokl_scr[...] = jnp.dot(
                tok_ref[...], w_ref[...], preferred_element_type=jnp.float32
            )

        prow = (
            jnp.dot(
                pos_ref[pl.ds(t, 1), :],
                w_ref[...],
                preferred_element_type=jnp.float32,
            )
            + b_ref[...]
        )
        out_ref[...] = tokl_scr[...] + prow

    return pl.pallas_call(
        body,
        grid=(tx,),
        in_specs=[
            pl.BlockSpec((voc, d), lambda i: (0, 0)),
            pl.BlockSpec((tx, d), lambda i: (0, 0)),
            pl.BlockSpec((d, vp), lambda i: (0, 0)),
            pl.BlockSpec((1, vp), lambda i: (0, 0)),
        ],
        out_specs=pl.BlockSpec((voc, vp), lambda i: (i, 0)),
        out_shape=jax.ShapeDtypeStruct((tx * voc, vp), jnp.float32),
        scratch_shapes=[pltpu.VMEM((voc, vp), jnp.float32)],
    )


def _sc_head(bx: int, tx: int, v: int, vp: int, txp: int):
    n_per_w = bx // _NW
    mesh = plsc.VectorSubcoreMesh(core_axis_name="c", subcore_axis_name="s")

    @functools.partial(
        pl.kernel,
        mesh=mesh,
        out_type=jax.ShapeDtypeStruct((bx, tx, v), jnp.float32),
        scratch_types=[
            pltpu.VMEM((n_per_w * txp,), jnp.int32),
            pltpu.VMEM((txp, vp), jnp.float32),
            pltpu.VMEM((tx, v), jnp.float32),
            pltpu.SemaphoreType.DMA,
            pltpu.SemaphoreType.DMA,
        ],
    )
    def k(idx_hbm, combo_hbm, out_hbm, idx_v, bufa, buf3, gsem, wsem):
        wid = lax.axis_index("s") * _NC + lax.axis_index("c")
        sbase = wid * n_per_w
        pltpu.sync_copy(idx_hbm.at[pl.ds(sbase * txp, n_per_w * txp)], idx_v)

        def gather(s):
            return pltpu.make_async_copy(
                combo_hbm.at[idx_v.at[pl.ds(s * txp, txp)]], bufa, gsem
            )

        def write(s):
            return pltpu.make_async_copy(buf3, out_hbm.at[sbase + s], wsem)

        def seq_body(s, carry):
            gather(s).start()
            gather(s).wait()

            nk = v // 16
            def row_body(t, c):
                for kk in range(nk):
                    sl = pl.ds(kk * 16, 16)
                    buf3[t, sl] = bufa[t, sl]
                tl = pl.ds(v - 16, 16)
                buf3[t, tl] = bufa[t, tl]
                return c

            lax.fori_loop(0, tx, row_body, 0)
            write(s).start()
            write(s).wait()
            return carry

        lax.fori_loop(0, n_per_w, seq_body, 0)

    return k


def kernel(x, tok_table, pos_table, W, b):
    bx, tx = x.shape
    vocab, d = tok_table.shape
    v = W.shape[1]
    vp = 1024
    txp = 56

    w_pad = jnp.pad(W, ((0, 0), (0, vp - v)))
    b_pad = jnp.pad(b, (0, vp - v)).reshape(1, vp)
    combo = _tc_combo(vocab, d, tx, vp)(tok_table, pos_table, w_pad, b_pad)

    flat = x.astype(jnp.int32) + jnp.arange(tx, dtype=jnp.int32)[None, :] * vocab
    idx = jnp.pad(flat, ((0, 0), (0, txp - tx))).reshape(-1)
    return _sc_head(bx, tx, v, vp, txp)(idx, combo)
